# Initial kernel scaffold; baseline (speedup 1.0000x reference)
import jax
import jax.numpy as jnp
from jax.experimental import pallas as pl


def kernel(x, gate_w, w1, b1, ln_g, ln_b, w2, b2, temperature, expert_usage):
    raise NotImplementedError("write your pallas kernel here")


# speedup vs baseline: 2.5057x; 2.5057x over previous
import functools

import jax
import jax.numpy as jnp
from jax.experimental import pallas as pl

B, D, E, H = 32768, 768, 64, 64
BM = 512


def _router_block(x_ref, a_ref, p_ref, o_ref):
    xb = x_ref[...]
    y = jnp.dot(xb, a_ref[...], preferred_element_type=jnp.float32)

    s = y[:, :E]
    h = y[:, E:] + p_ref[0, :H]

    mu = jnp.mean(h, axis=1, keepdims=True)
    var = jnp.mean((h - mu) * (h - mu), axis=1, keepdims=True)
    hn = (h - mu) / jnp.sqrt(var + 1e-5) * p_ref[1, :H] + p_ref[2, :H]
    hg = 0.5 * hn * (1.0 + jax.lax.erf(hn * 0.7071067811865476))

    cap_logit = jnp.sum(hg * p_ref[3, :H], axis=1, keepdims=True) + p_ref[5, 0]
    cap = jax.nn.sigmoid(cap_logit)

    g = (s + p_ref[4, :E]) * cap

    col = jax.lax.broadcasted_iota(jnp.int32, g.shape, 1)
    v1 = jnp.max(g, axis=1, keepdims=True)
    i1 = jnp.min(jnp.where(g == v1, col, E), axis=1, keepdims=True)
    m1 = col == i1
    gm = jnp.where(m1, -jnp.inf, g)
    v2 = jnp.max(gm, axis=1, keepdims=True)
    i2 = jnp.min(jnp.where(gm == v2, col, E), axis=1, keepdims=True)
    m2 = col == i2

    rw = jnp.where(m1, v1, jnp.where(m2, v2, 0.0))
    o_ref[...] = rw / (v1 + v2 + 1e-6)


@jax.jit
def _router(x, a, params):
    return pl.pallas_call(
        _router_block,
        grid=(B // BM,),
        in_specs=[
            pl.BlockSpec((BM, D), lambda i: (i, 0)),
            pl.BlockSpec((D, 2 * E), lambda i: (0, 0)),
            pl.BlockSpec((8, 2 * E), lambda i: (0, 0)),
        ],
        out_specs=pl.BlockSpec((BM, E), lambda i: (i, 0)),
        out_shape=jax.ShapeDtypeStruct((B, E), jnp.float32),
    )(x, a, params)


def kernel(x, gate_w, w1, b1, ln_g, ln_b, w2, b2, temperature, expert_usage):
    t = jnp.clip(temperature, 0.1, None)[0]
    lb = expert_usage / (jnp.sum(expert_usage) + 1e-6)
    a = jnp.concatenate([gate_w.T * t, w1.T], axis=1)
    bias = -0.1 * lb * t
    params = jnp.zeros((8, 2 * E), jnp.float32)
    params = params.at[0, :H].set(b1)
    params = params.at[1, :H].set(ln_g)
    params = params.at[2, :H].set(ln_b)
    params = params.at[3, :H].set(w2[0])
    params = params.at[4, :E].set(bias)
    params = params.at[5, 0].set(b2[0])
    return _router(x, a, params)
